# baseline (speedup 1.0000x reference)
import jax
import jax.numpy as jnp
from jax import lax
from jax.experimental import pallas as pl
from jax.experimental.pallas import tpu as pltpu
from jax.experimental.pallas import tpu_sc as plsc

N = 10000
E = 320000
D = 128

NT = 16
N_PAD = 10240
SLICE = N_PAD // NT
EC = E // NT
FULL_ROWS = EC // 128
TAIL = EC - FULL_ROWS * 128


def _matvec_body(x_ref, w_ref, o_ref):
    o_ref[...] = jnp.sum(x_ref[...] * w_ref[...], axis=1)


def _propagate_body(y_hbm, src_hbm, dst_hbm, w_hbm, out_hbm,
                    src_v, dst_v, w_v, msg_v, y_v, zer_v, acc1, acc2, sem):
    t = lax.axis_index("s")

    pltpu.sync_copy(src_hbm.at[pl.ds(t * EC, EC)], src_v)
    pltpu.sync_copy(dst_hbm.at[pl.ds(t * EC, EC)], dst_v)
    pltpu.sync_copy(w_hbm.at[pl.ds(t * EC, EC)], w_v)
    pltpu.sync_copy(y_hbm, y_v)

    for i in range(SLICE // 16):
        zer_v[pl.ds(i * 16, 16)] = jnp.zeros((16,), jnp.float32)
    pltpu.sync_copy(zer_v, acc1.at[pl.ds(t * SLICE, SLICE)])
    pltpu.sync_copy(zer_v, acc2.at[pl.ds(t * SLICE, SLICE)])
    plsc.subcore_barrier()

    def do_round(acc):
        def row(r, _):
            base = r * 128
            for c in range(8):
                s16 = src_v[pl.ds(base + c * 16, 16)]
                w16 = w_v[pl.ds(base + c * 16, 16)]
                g16 = plsc.load_gather(y_v, [s16])
                msg_v[pl.ds(base + c * 16, 16)] = w16 * g16
            pltpu.async_copy(msg_v.at[pl.ds(base, 128)],
                             acc.at[dst_v.at[pl.ds(base, 128)]], sem,
                             add=True)
            return 0
        lax.fori_loop(0, FULL_ROWS, row, 0)

        tb = FULL_ROWS * 128
        for c in range(TAIL // 16):
            s16 = src_v[pl.ds(tb + c * 16, 16)]
            w16 = w_v[pl.ds(tb + c * 16, 16)]
            g16 = plsc.load_gather(y_v, [s16])
            msg_v[pl.ds(tb + c * 16, 16)] = w16 * g16
        pltpu.async_copy(msg_v.at[pl.ds(tb, TAIL)],
                         acc.at[dst_v.at[pl.ds(tb, TAIL)]], sem, add=True)

        def drain(r, _):
            pltpu.make_async_copy(msg_v.at[pl.ds(r * 128, 128)],
                                  acc.at[dst_v.at[pl.ds(r * 128, 128)]],
                                  sem).wait()
            return 0
        lax.fori_loop(0, FULL_ROWS, drain, 0)
        pltpu.make_async_copy(msg_v.at[pl.ds(tb, TAIL)],
                              acc.at[dst_v.at[pl.ds(tb, TAIL)]],
                              sem).wait()

    do_round(acc1)
    plsc.subcore_barrier()

    pltpu.sync_copy(acc1, y_v)
    do_round(acc2)
    plsc.subcore_barrier()

    pltpu.sync_copy(acc2.at[pl.ds(t * SLICE, SLICE)],
                    out_hbm.at[pl.ds(t * SLICE, SLICE)])


def kernel(x, edge_index, edge_weight, W, b):
    y_pad = pl.pallas_call(
        _matvec_body,
        grid=(N_PAD // 1024,),
        in_specs=[
            pl.BlockSpec((1024, D), lambda i: (i, 0)),
            pl.BlockSpec((1, D), lambda i: (0, 0)),
        ],
        out_specs=pl.BlockSpec((1024,), lambda i: (i,)),
        out_shape=jax.ShapeDtypeStruct((N_PAD,), jnp.float32),
    )(x, W)

    src = edge_index[0]
    dst = edge_index[1]

    mesh = plsc.VectorSubcoreMesh(
        core_axis_name="c", subcore_axis_name="s", num_cores=1)
    propagate = pl.kernel(
        _propagate_body,
        out_type=jax.ShapeDtypeStruct((N_PAD,), jnp.float32),
        mesh=mesh,
        scratch_types=[
            pltpu.VMEM((EC,), jnp.int32),
            pltpu.VMEM((EC,), jnp.int32),
            pltpu.VMEM((EC,), jnp.float32),
            pltpu.VMEM((EC,), jnp.float32),
            pltpu.VMEM((N_PAD,), jnp.float32),
            pltpu.VMEM((SLICE,), jnp.float32),
            pltpu.VMEM_SHARED((N_PAD,), jnp.float32),
            pltpu.VMEM_SHARED((N_PAD,), jnp.float32),
            pltpu.SemaphoreType.DMA,
        ],
        compiler_params=pltpu.CompilerParams(needs_layout_passes=False),
    )
    y2 = propagate(y_pad, src, dst, edge_weight)

    return y2[:N, None] + b


# speedup vs baseline: 1.0269x; 1.0269x over previous
import jax
import jax.numpy as jnp
from jax import lax
from jax.experimental import pallas as pl
from jax.experimental.pallas import tpu as pltpu
from jax.experimental.pallas import tpu_sc as plsc

N = 10000
E = 320000
D = 128

NT = 16
N_PAD = 10240
SLICE = N_PAD // NT
EC = E // NT
FULL_ROWS = EC // 128
TAIL = EC - FULL_ROWS * 128


def _matvec_body(w_ref, x_ref, o_ref):
    o_ref[...] = lax.dot_general(
        w_ref[...], x_ref[...], (((1,), (1,)), ((), ())))


def _propagate_body(y_hbm, src_hbm, dst_hbm, w_hbm, out_hbm,
                    src_v, dst_v, w_v, msg_v, y_v, zer_v, acc1, acc2, sem):
    t = lax.axis_index("s")

    pltpu.sync_copy(src_hbm.at[pl.ds(t * EC, EC)], src_v)
    pltpu.sync_copy(dst_hbm.at[pl.ds(t * EC, EC)], dst_v)
    pltpu.sync_copy(w_hbm.at[pl.ds(t * EC, EC)], w_v)
    pltpu.sync_copy(y_hbm.at[0], y_v)

    for i in range(SLICE // 16):
        zer_v[pl.ds(i * 16, 16)] = jnp.zeros((16,), jnp.float32)
    pltpu.sync_copy(zer_v, acc1.at[pl.ds(t * SLICE, SLICE)])
    pltpu.sync_copy(zer_v, acc2.at[pl.ds(t * SLICE, SLICE)])
    plsc.subcore_barrier()

    def do_round(acc):
        def row(r, _):
            base = r * 128
            for c in range(8):
                s16 = src_v[pl.ds(base + c * 16, 16)]
                w16 = w_v[pl.ds(base + c * 16, 16)]
                g16 = plsc.load_gather(y_v, [s16])
                msg_v[pl.ds(base + c * 16, 16)] = w16 * g16
            pltpu.async_copy(msg_v.at[pl.ds(base, 128)],
                             acc.at[dst_v.at[pl.ds(base, 128)]], sem,
                             add=True)
            return 0
        lax.fori_loop(0, FULL_ROWS, row, 0)

        tb = FULL_ROWS * 128
        for c in range(TAIL // 16):
            s16 = src_v[pl.ds(tb + c * 16, 16)]
            w16 = w_v[pl.ds(tb + c * 16, 16)]
            g16 = plsc.load_gather(y_v, [s16])
            msg_v[pl.ds(tb + c * 16, 16)] = w16 * g16
        pltpu.async_copy(msg_v.at[pl.ds(tb, TAIL)],
                         acc.at[dst_v.at[pl.ds(tb, TAIL)]], sem, add=True)

        def drain(r, _):
            pltpu.make_async_copy(msg_v.at[pl.ds(r * 128, 128)],
                                  acc.at[dst_v.at[pl.ds(r * 128, 128)]],
                                  sem).wait()
            return 0
        lax.fori_loop(0, FULL_ROWS, drain, 0)
        pltpu.make_async_copy(msg_v.at[pl.ds(tb, TAIL)],
                              acc.at[dst_v.at[pl.ds(tb, TAIL)]],
                              sem).wait()

    do_round(acc1)
    plsc.subcore_barrier()

    pltpu.sync_copy(acc1, y_v)
    do_round(acc2)
    plsc.subcore_barrier()

    pltpu.sync_copy(acc2.at[pl.ds(t * SLICE, SLICE)],
                    out_hbm.at[pl.ds(t * SLICE, SLICE)])


def kernel(x, edge_index, edge_weight, W, b):
    y_pad = pl.pallas_call(
        _matvec_body,
        grid=(N_PAD // 1024,),
        in_specs=[
            pl.BlockSpec((1, D), lambda i: (0, 0)),
            pl.BlockSpec((1024, D), lambda i: (i, 0)),
        ],
        out_specs=pl.BlockSpec((1, 1024), lambda i: (0, i)),
        out_shape=jax.ShapeDtypeStruct((1, N_PAD), jnp.float32),
    )(W, x)

    src = edge_index[0]
    dst = edge_index[1]

    mesh = plsc.VectorSubcoreMesh(
        core_axis_name="c", subcore_axis_name="s", num_cores=1)
    propagate = pl.kernel(
        _propagate_body,
        out_type=jax.ShapeDtypeStruct((N_PAD,), jnp.float32),
        mesh=mesh,
        scratch_types=[
            pltpu.VMEM((EC,), jnp.int32),
            pltpu.VMEM((EC,), jnp.int32),
            pltpu.VMEM((EC,), jnp.float32),
            pltpu.VMEM((EC,), jnp.float32),
            pltpu.VMEM((N_PAD,), jnp.float32),
            pltpu.VMEM((SLICE,), jnp.float32),
            pltpu.VMEM_SHARED((N_PAD,), jnp.float32),
            pltpu.VMEM_SHARED((N_PAD,), jnp.float32),
            pltpu.SemaphoreType.DMA,
        ],
        compiler_params=pltpu.CompilerParams(needs_layout_passes=False),
    )
    y2 = propagate(y_pad, src, dst, edge_weight)

    return y2[:N, None] + b


# speedup vs baseline: 1.1598x; 1.1293x over previous
import jax
import jax.numpy as jnp
from jax import lax
from jax.experimental import pallas as pl
from jax.experimental.pallas import tpu as pltpu
from jax.experimental.pallas import tpu_sc as plsc

N = 10000
E = 320000
D = 128

NT = 16
N_PAD = 10240
SLICE = N_PAD // NT
EC = E // NT
FULL_ROWS = EC // 128
TAIL = EC - FULL_ROWS * 128


def _matvec_body(w_ref, x_ref, o_ref):
    o_ref[...] = lax.dot_general(
        w_ref[...], x_ref[...], (((1,), (1,)), ((), ())))


def _propagate_body(y_hbm, e_hbm, w_hbm, out_hbm,
                    src_v, dst_v, w_v, msg_v, y_v, zer_v, acc1, acc2, sem):
    t = lax.axis_index("s")

    pltpu.sync_copy(e_hbm.at[0, t], src_v)
    pltpu.sync_copy(e_hbm.at[1, t], dst_v)
    pltpu.sync_copy(w_hbm.at[t], w_v)
    pltpu.sync_copy(y_hbm.at[0], y_v)

    for i in range(SLICE // 16):
        zer_v[pl.ds(i * 16, 16)] = jnp.zeros((16,), jnp.float32)
    pltpu.sync_copy(zer_v, acc1.at[pl.ds(t * SLICE, SLICE)])
    pltpu.sync_copy(zer_v, acc2.at[pl.ds(t * SLICE, SLICE)])
    plsc.subcore_barrier()

    def do_round(acc):
        def row(r, _):
            base = r * 128
            for c in range(8):
                s16 = src_v[pl.ds(base + c * 16, 16)]
                w16 = w_v[pl.ds(base + c * 16, 16)]
                g16 = plsc.load_gather(y_v, [s16])
                msg_v[pl.ds(base + c * 16, 16)] = w16 * g16
            pltpu.async_copy(msg_v.at[pl.ds(base, 128)],
                             acc.at[dst_v.at[pl.ds(base, 128)]], sem,
                             add=True)
            return 0
        lax.fori_loop(0, FULL_ROWS, row, 0)

        tb = FULL_ROWS * 128
        for c in range(TAIL // 16):
            s16 = src_v[pl.ds(tb + c * 16, 16)]
            w16 = w_v[pl.ds(tb + c * 16, 16)]
            g16 = plsc.load_gather(y_v, [s16])
            msg_v[pl.ds(tb + c * 16, 16)] = w16 * g16
        pltpu.async_copy(msg_v.at[pl.ds(tb, TAIL)],
                         acc.at[dst_v.at[pl.ds(tb, TAIL)]], sem, add=True)

        def drain(r, _):
            pltpu.make_async_copy(msg_v.at[pl.ds(r * 128, 128)],
                                  acc.at[dst_v.at[pl.ds(r * 128, 128)]],
                                  sem).wait()
            return 0
        lax.fori_loop(0, FULL_ROWS, drain, 0)
        pltpu.make_async_copy(msg_v.at[pl.ds(tb, TAIL)],
                              acc.at[dst_v.at[pl.ds(tb, TAIL)]],
                              sem).wait()

    do_round(acc1)
    plsc.subcore_barrier()

    pltpu.sync_copy(acc1, y_v)
    do_round(acc2)
    plsc.subcore_barrier()

    pltpu.sync_copy(acc2.at[pl.ds(t * SLICE, SLICE)],
                    out_hbm.at[pl.ds(t * SLICE, SLICE)])


def kernel(x, edge_index, edge_weight, W, b):
    y_pad = pl.pallas_call(
        _matvec_body,
        grid=(N_PAD // 1024,),
        in_specs=[
            pl.BlockSpec((1, D), lambda i: (0, 0)),
            pl.BlockSpec((1024, D), lambda i: (i, 0)),
        ],
        out_specs=pl.BlockSpec((1, 1024), lambda i: (0, i)),
        out_shape=jax.ShapeDtypeStruct((1, N_PAD), jnp.float32),
    )(W, x)

    e3 = edge_index.reshape(2, NT, EC)
    w2 = edge_weight.reshape(NT, EC)

    mesh = plsc.VectorSubcoreMesh(
        core_axis_name="c", subcore_axis_name="s", num_cores=1)
    propagate = pl.kernel(
        _propagate_body,
        out_type=jax.ShapeDtypeStruct((N_PAD,), jnp.float32),
        mesh=mesh,
        scratch_types=[
            pltpu.VMEM((EC,), jnp.int32),
            pltpu.VMEM((EC,), jnp.int32),
            pltpu.VMEM((EC,), jnp.float32),
            pltpu.VMEM((EC,), jnp.float32),
            pltpu.VMEM((N_PAD,), jnp.float32),
            pltpu.VMEM((SLICE,), jnp.float32),
            pltpu.VMEM_SHARED((N_PAD,), jnp.float32),
            pltpu.VMEM_SHARED((N_PAD,), jnp.float32),
            pltpu.SemaphoreType.DMA,
        ],
        compiler_params=pltpu.CompilerParams(needs_layout_passes=False),
    )
    y2 = propagate(y_pad, e3, w2)

    return y2[:N, None] + b
